# baseline (speedup 1.0000x reference)
import functools

import jax
import jax.numpy as jnp
from jax import lax
from jax.experimental import pallas as pl
from jax.experimental.pallas import tpu as pltpu
from jax.experimental.pallas import tpu_sc as plsc

_B = 256
_D = 256
_K = 1024


def _dist_kernel(x_ref, w_ref, idx_ref, loss_ref):
    x = x_ref[...] + 1e-6
    w = w_ref[...]
    wsq_half = 0.5 * jnp.sum(w * w, axis=0, keepdims=True)
    xw = jax.lax.dot_general(
        x, w, (((1,), (0,)), ((), ())),
        preferred_element_type=jnp.float32,
        precision=jax.lax.Precision.HIGHEST,
    )
    s = wsq_half - xw
    min_s = jnp.min(s, axis=1)
    idx_ref[...] = jnp.argmin(s, axis=1)
    xsq = jnp.sum(x * x, axis=1)
    d2min = jnp.maximum(xsq + 2.0 * min_s, 0.0)
    loss_ref[...] = jnp.reshape(
        jnp.sum(jnp.sqrt(d2min)) / jnp.float32(_B), (1, 1))


_SC_MESH = plsc.VectorSubcoreMesh(core_axis_name="c", subcore_axis_name="s")


@functools.partial(
    pl.kernel,
    mesh=_SC_MESH,
    compiler_params=pltpu.CompilerParams(needs_layout_passes=False),
    out_type=jax.ShapeDtypeStruct((2 * _B,), jnp.float32),
    scratch_types=[
        pltpu.VMEM((2 * _K,), jnp.float32),
        pltpu.VMEM((_B,), jnp.int32),
        pltpu.VMEM((16,), jnp.float32),
    ],
)
def _gather_kernel(loc_hbm, idx_hbm, out_hbm, loc_v, idx_v, out_v):
    wid = lax.axis_index("s") * 2 + lax.axis_index("c")
    pltpu.sync_copy(loc_hbm, loc_v)
    pltpu.sync_copy(idx_hbm, idx_v)
    j = 16 * wid + lax.iota(jnp.int32, 16)
    bmu = plsc.load_gather(idx_v, [j >> 1])
    out_v[...] = plsc.load_gather(loc_v, [2 * bmu + (j & 1)])
    pltpu.sync_copy(out_v, out_hbm.at[pl.ds(16 * wid, 16)])


def kernel(input, weight, locations):
    idx, loss = pl.pallas_call(
        _dist_kernel,
        out_shape=(
            jax.ShapeDtypeStruct((_B,), jnp.int32),
            jax.ShapeDtypeStruct((1, 1), jnp.float32),
        ),
    )(input, weight)
    bmu_flat = _gather_kernel(locations.reshape(-1), idx)
    return bmu_flat.reshape(_B, 1, 2), loss.reshape(())


# speedup vs baseline: 2.4032x; 2.4032x over previous
import jax
import jax.numpy as jnp
from jax.experimental import pallas as pl
from jax.experimental.pallas import tpu as pltpu

_B = 256
_D = 256
_K = 1024
_G = 4
_KB = _K // _G


def _som_kernel(x_ref, w_ref, loc_ref, bmu_ref, loss_ref, best_ref, bi_ref):
    g = pl.program_id(0)
    x = x_ref[...] + 1e-6
    w = w_ref[...]
    wsq_half = 0.5 * jnp.sum(w * w, axis=0, keepdims=True)
    xw = jax.lax.dot_general(
        x, w, (((1,), (0,)), ((), ())),
        preferred_element_type=jnp.float32,
        precision=jax.lax.Precision.HIGHEST,
    )
    s = wsq_half - xw
    m = jnp.min(s, axis=1)
    a = jnp.argmin(s, axis=1) + g * _KB

    @pl.when(g == 0)
    def _init():
        best_ref[...] = m
        bi_ref[...] = a

    @pl.when(g > 0)
    def _update():
        upd = m < best_ref[...]
        bi_ref[...] = jnp.where(upd, a, bi_ref[...])
        best_ref[...] = jnp.where(upd, m, best_ref[...])

    @pl.when(g == _G - 1)
    def _finish():
        xsq = jnp.sum(x * x, axis=1)
        d2min = jnp.maximum(xsq + 2.0 * best_ref[...], 0.0)
        loss_ref[...] = jnp.reshape(
            jnp.sum(jnp.sqrt(d2min)) / jnp.float32(_B), (1, 1))
        onehot = (jax.lax.broadcasted_iota(jnp.int32, (_B, _K), 1)
                  == bi_ref[...][:, None]).astype(jnp.float32)
        bmu_ref[...] = jax.lax.dot_general(
            onehot, loc_ref[...], (((1,), (0,)), ((), ())),
            preferred_element_type=jnp.float32,
        )


def kernel(input, weight, locations):
    bmu, loss = pl.pallas_call(
        _som_kernel,
        grid=(_G,),
        in_specs=[
            pl.BlockSpec((_B, _D), lambda g: (0, 0)),
            pl.BlockSpec((_D, _KB), lambda g: (0, g)),
            pl.BlockSpec((_K, 2), lambda g: (0, 0)),
        ],
        out_specs=(
            pl.BlockSpec((_B, 2), lambda g: (0, 0)),
            pl.BlockSpec((1, 1), lambda g: (0, 0)),
        ),
        out_shape=(
            jax.ShapeDtypeStruct((_B, 2), jnp.float32),
            jax.ShapeDtypeStruct((1, 1), jnp.float32),
        ),
        scratch_shapes=[
            pltpu.VMEM((_B,), jnp.float32),
            pltpu.VMEM((_B,), jnp.int32),
        ],
    )(input, weight, locations)
    return bmu.reshape(_B, 1, 2), loss.reshape(())


# speedup vs baseline: 2.8767x; 1.1970x over previous
import jax
import jax.numpy as jnp
from jax.experimental import pallas as pl
from jax.experimental.pallas import tpu as pltpu

_B = 256
_D = 256
_K = 1024
_H = _K // 2


def _half(x, w):
    wsq_half = 0.5 * jnp.sum(w * w, axis=0, keepdims=True)
    xw = jax.lax.dot_general(
        x, w, (((1,), (0,)), ((), ())),
        preferred_element_type=jnp.float32,
        precision=jax.lax.Precision.HIGHEST,
    )
    s = wsq_half - xw
    return jnp.min(s, axis=1), jnp.argmin(s, axis=1)


def _som_kernel(x_ref, w_hbm, loc_ref, bmu_ref, loss_ref, w_v, sem):
    cp0 = pltpu.make_async_copy(w_hbm.at[:, pl.ds(0, _H)],
                                w_v.at[:, pl.ds(0, _H)], sem.at[0])
    cp1 = pltpu.make_async_copy(w_hbm.at[:, pl.ds(_H, _H)],
                                w_v.at[:, pl.ds(_H, _H)], sem.at[1])
    cp0.start()
    cp1.start()
    x = x_ref[...] + 1e-6
    xsq = jnp.sum(x * x, axis=1)
    cp0.wait()
    m0, a0 = _half(x, w_v[:, pl.ds(0, _H)])
    cp1.wait()
    m1, a1 = _half(x, w_v[:, pl.ds(_H, _H)])
    upd = m1 < m0
    best = jnp.where(upd, m1, m0)
    idx = jnp.where(upd, a1 + _H, a0)
    d2min = jnp.maximum(xsq + 2.0 * best, 0.0)
    loss_ref[...] = jnp.reshape(
        jnp.sum(jnp.sqrt(d2min)) / jnp.float32(_B), (1, 1))
    onehot = (jax.lax.broadcasted_iota(jnp.int32, (_B, _K), 1)
              == idx[:, None]).astype(jnp.float32)
    bmu_ref[...] = jax.lax.dot_general(
        onehot, loc_ref[...], (((1,), (0,)), ((), ())),
        preferred_element_type=jnp.float32,
    )


def kernel(input, weight, locations):
    bmu, loss = pl.pallas_call(
        _som_kernel,
        in_specs=[
            pl.BlockSpec(memory_space=pltpu.VMEM),
            pl.BlockSpec(memory_space=pltpu.HBM),
            pl.BlockSpec(memory_space=pltpu.VMEM),
        ],
        out_specs=(
            pl.BlockSpec(memory_space=pltpu.VMEM),
            pl.BlockSpec(memory_space=pltpu.VMEM),
        ),
        out_shape=(
            jax.ShapeDtypeStruct((_B, 2), jnp.float32),
            jax.ShapeDtypeStruct((1, 1), jnp.float32),
        ),
        scratch_shapes=[
            pltpu.VMEM((_D, _K), jnp.float32),
            pltpu.SemaphoreType.DMA((2,)),
        ],
    )(input, weight, locations)
    return bmu.reshape(_B, 1, 2), loss.reshape(())


# speedup vs baseline: 3.3604x; 1.1681x over previous
import jax
import jax.numpy as jnp
from jax.experimental import pallas as pl

_B = 256
_D = 256
_K = 1024


def _som_kernel(x_ref, w_ref, loc_ref, bmu_ref, loss_ref):
    x = x_ref[...] + 1e-6
    w = w_ref[...]
    wsq_half = 0.5 * jnp.sum(w * w, axis=0, keepdims=True)
    xw = jax.lax.dot_general(
        x, w, (((1,), (0,)), ((), ())),
        preferred_element_type=jnp.float32,
        precision=jax.lax.Precision.HIGHEST,
    )
    s = wsq_half - xw
    min_s = jnp.min(s, axis=1)
    idx = jnp.argmin(s, axis=1)
    xsq = jnp.sum(x * x, axis=1)
    d2min = jnp.maximum(xsq + 2.0 * min_s, 0.0)
    loss_ref[...] = jnp.reshape(
        jnp.sum(jnp.sqrt(d2min)) / jnp.float32(_B), (1, 1))
    onehot = (jax.lax.broadcasted_iota(jnp.int32, (_B, _K), 1)
              == idx[:, None]).astype(jnp.bfloat16)
    bmu_ref[...] = jax.lax.dot_general(
        onehot, loc_ref[...].astype(jnp.bfloat16), (((1,), (0,)), ((), ())),
        preferred_element_type=jnp.float32,
    )


def kernel(input, weight, locations):
    bmu, loss = pl.pallas_call(
        _som_kernel,
        out_shape=(
            jax.ShapeDtypeStruct((_B, 2), jnp.float32),
            jax.ShapeDtypeStruct((1, 1), jnp.float32),
        ),
    )(input, weight, locations)
    return bmu.reshape(_B, 1, 2), loss.reshape(())
